# baseline (speedup 1.0000x reference)
import functools

import jax
import jax.numpy as jnp
from jax import lax
from jax.experimental import pallas as pl
from jax.experimental.pallas import tpu as pltpu
from jax.experimental.pallas import tpu_sc as plsc

NC = 2
NS = 16
NW = NC * NS
L = 16
G = 128


@functools.lru_cache(maxsize=None)
def _build(N, S, V, D):
    b_per_w = N // NW
    ng = b_per_w // G
    n2 = S // 2
    assert b_per_w % G == 0 and S % 2 == 0 and D == 2 * L

    mesh = plsc.VectorSubcoreMesh(
        core_axis_name="c", subcore_axis_name="s",
        num_cores=NC, num_subcores=NS)

    @functools.partial(
        pl.kernel,
        out_type=jax.ShapeDtypeStruct((S, D, N), jnp.float32),
        mesh=mesh,
        scratch_types=[
            pltpu.VMEM((S, b_per_w), jnp.int32),
            pltpu.VMEM((b_per_w, D), jnp.float32),
            pltpu.VMEM((b_per_w, D), jnp.float32),
            pltpu.VMEM((D, b_per_w), jnp.float32),
            pltpu.VMEM((D, b_per_w), jnp.float32),
            pltpu.SemaphoreType.DMA,
            pltpu.SemaphoreType.DMA,
            pltpu.SemaphoreType.DMA,
            pltpu.SemaphoreType.DMA,
        ],
        compiler_params=pltpu.CompilerParams(
            use_tc_tiling_on_sc=False, needs_layout_passes=False),
    )
    def emb(idx_hbm, table_hbm, out_hbm, idx_v, buf0, buf1, bufT0, bufT1,
            gsem0, gsem1, osem0, osem1):
        wid = lax.axis_index("s") * NC + lax.axis_index("c")
        b0 = wid * b_per_w

        pltpu.sync_copy(idx_hbm.at[pl.ds(0, S), pl.ds(b0, b_per_w)], idx_v)

        def fire_gathers(a, buf, sem):
            for k in range(ng):
                pltpu.async_copy(
                    table_hbm.at[idx_v.at[a, pl.ds(k * G, G)]],
                    buf.at[pl.ds(k * G, G)], sem)

        def wait_gathers(a, buf, sem):
            for k in range(ng):
                pltpu.make_async_copy(
                    table_hbm.at[idx_v.at[a, pl.ds(k * G, G)]],
                    buf.at[pl.ds(k * G, G)], sem).wait()

        def fire_scatter(a, bufT, sem):
            pltpu.async_copy(
                bufT, out_hbm.at[a, pl.ds(0, D), pl.ds(b0, b_per_w)], sem)

        def wait_scatter(a, bufT, sem):
            pltpu.make_async_copy(
                bufT, out_hbm.at[a, pl.ds(0, D), pl.ds(b0, b_per_w)],
                sem).wait()

        def transpose(buf, bufT):
            iota0 = lax.iota(jnp.int32, L)
            cvecs = [jnp.full((L,), c, jnp.int32) for c in range(D)]

            def tbody(g, rvec):
                base = g * L
                for c in range(D):
                    v = plsc.load_gather(buf, [rvec, cvecs[c]])
                    bufT[c, pl.ds(base, L)] = v
                return rvec + L
            lax.fori_loop(0, b_per_w // L, tbody, iota0)

        def fix_zeros(a, buf):
            zany = jnp.zeros((L,), jnp.int32)
            for j in range(b_per_w // L):
                v = idx_v[a, pl.ds(j * L, L)]
                zany = zany | (v == 0).astype(jnp.int32)
            nz = zany[0]
            for k in range(1, L):
                nz = nz + zany[k]

            @pl.when(nz > 0)
            def _():
                zero = jnp.zeros((L,), jnp.float32)
                for j in range(b_per_w // L):
                    v = idx_v[a, pl.ds(j * L, L)]
                    msk = v == 0
                    rows = j * L + lax.iota(jnp.int32, L)

                    def fbody(f, fvec):
                        plsc.store_scatter(buf, [rows, fvec], zero, mask=msk)
                        return fvec + 1
                    lax.fori_loop(0, D, fbody, jnp.zeros((L,), jnp.int32))

        fire_gathers(0, buf0, gsem0)

        def body(i, carry):
            a = 2 * i
            b = a + 1
            wait_gathers(a, buf0, gsem0)
            fix_zeros(a, buf0)

            @pl.when(i > 0)
            def _():
                wait_scatter(b - 2, bufT1, osem1)
            fire_gathers(b, buf1, gsem1)
            transpose(buf0, bufT0)
            fire_scatter(a, bufT0, osem0)

            wait_gathers(b, buf1, gsem1)
            fix_zeros(b, buf1)
            wait_scatter(a, bufT0, osem0)

            @pl.when(i < n2 - 1)
            def _():
                fire_gathers(a + 2, buf0, gsem0)
            transpose(buf1, bufT1)
            fire_scatter(b, bufT1, osem1)
            return carry

        lax.fori_loop(0, n2, body, 0)
        wait_scatter(2 * n2 - 1, bufT1, osem1)

    return emb


def kernel(inputs, table):
    V, D = table.shape
    N, S = inputs.shape
    out_t = _build(N, S, V, D)(inputs.T.astype(jnp.int32), table)
    return jnp.transpose(out_t, (2, 0, 1))


# speedup vs baseline: 1.4751x; 1.4751x over previous
import functools

import jax
import jax.numpy as jnp
from jax import lax
from jax.experimental import pallas as pl
from jax.experimental.pallas import tpu as pltpu
from jax.experimental.pallas import tpu_sc as plsc

NC = 2
NS = 16
NW = NC * NS
L = 16
G = 128


@functools.lru_cache(maxsize=None)
def _build(N, S, V, D):
    b_per_w = N // NW
    ng = b_per_w // G
    n2 = S // 2
    assert b_per_w % G == 0 and S % 2 == 0 and D == 2 * L

    mesh = plsc.VectorSubcoreMesh(
        core_axis_name="c", subcore_axis_name="s",
        num_cores=NC, num_subcores=NS)

    @functools.partial(
        pl.kernel,
        out_type=jax.ShapeDtypeStruct((S, D, N), jnp.float32),
        mesh=mesh,
        scratch_types=[
            pltpu.VMEM((S, b_per_w), jnp.int32),
            pltpu.VMEM((b_per_w, D), jnp.float32),
            pltpu.VMEM((b_per_w, D), jnp.float32),
            pltpu.VMEM((D, b_per_w), jnp.float32),
            pltpu.VMEM((D, b_per_w), jnp.float32),
            pltpu.SemaphoreType.DMA,
            pltpu.SemaphoreType.DMA,
            pltpu.SemaphoreType.DMA,
            pltpu.SemaphoreType.DMA,
        ],
        compiler_params=pltpu.CompilerParams(
            use_tc_tiling_on_sc=False, needs_layout_passes=False),
    )
    def emb(idx_hbm, table_hbm, out_hbm, idx_v, buf0, buf1, bufT0, bufT1,
            gsem0, gsem1, osem0, osem1):
        wid = lax.axis_index("s") * NC + lax.axis_index("c")
        b0 = wid * b_per_w

        pltpu.sync_copy(idx_hbm.at[pl.ds(0, S), pl.ds(b0, b_per_w)], idx_v)

        def fire_gathers(a, buf, sem):
            for k in range(ng):
                pltpu.async_copy(
                    table_hbm.at[idx_v.at[a, pl.ds(k * G, G)]],
                    buf.at[pl.ds(k * G, G)], sem)

        def wait_gathers(a, buf, sem):
            for k in range(ng):
                pltpu.make_async_copy(
                    table_hbm.at[idx_v.at[a, pl.ds(k * G, G)]],
                    buf.at[pl.ds(k * G, G)], sem).wait()

        def fire_scatter(a, bufT, sem):
            pltpu.async_copy(
                bufT, out_hbm.at[a, pl.ds(0, D), pl.ds(b0, b_per_w)], sem)

        def wait_scatter(a, bufT, sem):
            pltpu.make_async_copy(
                bufT, out_hbm.at[a, pl.ds(0, D), pl.ds(b0, b_per_w)],
                sem).wait()

        def transpose(buf, bufT):
            iota0 = lax.iota(jnp.int32, L)
            cvecs = [(iota0 + d) % D for d in range(D)]

            def tbody(g, rvec):
                for d in range(D):
                    v = plsc.load_gather(buf, [rvec, cvecs[d]])
                    plsc.store_scatter(bufT, [cvecs[d], rvec], v)
                return rvec + L
            lax.fori_loop(0, b_per_w // L, tbody, iota0)

        def fix_zeros(a, buf):
            zany = jnp.zeros((L,), jnp.int32)
            for j in range(b_per_w // L):
                v = idx_v[a, pl.ds(j * L, L)]
                zany = zany | (v == 0).astype(jnp.int32)
            nz = zany[0]
            for k in range(1, L):
                nz = nz + zany[k]

            @pl.when(nz > 0)
            def _():
                zero = jnp.zeros((L,), jnp.float32)
                for j in range(b_per_w // L):
                    v = idx_v[a, pl.ds(j * L, L)]
                    msk = v == 0
                    rows = j * L + lax.iota(jnp.int32, L)

                    def fbody(f, fvec):
                        plsc.store_scatter(buf, [rows, fvec], zero, mask=msk)
                        return fvec + 1
                    lax.fori_loop(0, D, fbody, jnp.zeros((L,), jnp.int32))

        fire_gathers(0, buf0, gsem0)

        def body(i, carry):
            a = 2 * i
            b = a + 1
            wait_gathers(a, buf0, gsem0)
            fix_zeros(a, buf0)

            @pl.when(i > 0)
            def _():
                wait_scatter(b - 2, bufT1, osem1)
            fire_gathers(b, buf1, gsem1)
            transpose(buf0, bufT0)
            fire_scatter(a, bufT0, osem0)

            wait_gathers(b, buf1, gsem1)
            fix_zeros(b, buf1)
            wait_scatter(a, bufT0, osem0)

            @pl.when(i < n2 - 1)
            def _():
                fire_gathers(a + 2, buf0, gsem0)
            transpose(buf1, bufT1)
            fire_scatter(b, bufT1, osem1)
            return carry

        lax.fori_loop(0, n2, body, 0)
        wait_scatter(2 * n2 - 1, bufT1, osem1)

    return emb


def kernel(inputs, table):
    V, D = table.shape
    N, S = inputs.shape
    out_t = _build(N, S, V, D)(inputs.T.astype(jnp.int32), table)
    return jnp.transpose(out_t, (2, 0, 1))
